# baseline (speedup 1.0000x reference)
import functools

import jax
import jax.numpy as jnp
from jax.experimental import pallas as pl
from jax.experimental.pallas import tpu as pltpu


def _fused_moe_kernel(x_ref, wr_ref, gu_ref, dn_ref,
                      out_ref, scores_ref, sc_scr, xb_scr,
                      *, top_k, n_exp, inter, tm):
    e = pl.program_id(1)
    ti = pl.program_id(2)
    rows = pl.ds(ti * tm, tm)
    x = x_ref[...]

    @pl.when(e == 0)
    def _router_and_shared():
        logits = jnp.dot(x, wr_ref[...], preferred_element_type=jnp.float32)
        col = jax.lax.broadcasted_iota(jnp.int32, logits.shape, 1)
        masked = logits
        keep = jnp.zeros(logits.shape, dtype=jnp.bool_)
        for _ in range(top_k):
            m = jnp.max(masked, axis=-1, keepdims=True)
            is_max = masked == m
            first_idx = jnp.min(jnp.where(is_max, col, n_exp),
                                axis=-1, keepdims=True)
            sel = col == first_idx
            keep = jnp.logical_or(keep, sel)
            masked = jnp.where(sel, -jnp.inf, masked)
        scores = jnp.where(keep, jax.nn.sigmoid(logits), 0.0)
        sc_scr[rows, :] = scores
        scores_ref[rows, :] = scores
        xb = x.astype(jnp.bfloat16)
        xb_scr[rows, :] = xb

        gu = jnp.dot(xb, gu_ref[...], preferred_element_type=jnp.float32)
        g = gu[:, :inter]
        u = gu[:, inter:]
        h = g * jax.nn.sigmoid(g) * u
        out_ref[rows, :] = jnp.dot(h, dn_ref[...],
                                   preferred_element_type=jnp.float32)

    @pl.when(e > 0)
    def _routed():
        half = tm // 2
        for hh in range(2):
            r2 = pl.ds(ti * tm + hh * half, half)
            sc = sc_scr[r2, :]
            col = jax.lax.broadcasted_iota(jnp.int32, sc.shape, 1)
            s = jnp.sum(jnp.where(col == e - 1, sc, 0.0), axis=1,
                        keepdims=True)
            gu = jnp.dot(xb_scr[r2, :], gu_ref[...],
                         preferred_element_type=jnp.float32) * s
            g = gu[:, :inter]
            u = gu[:, inter:]
            h = g * jax.nn.sigmoid(g) * u
            out_ref[r2, :] += jnp.dot(h, dn_ref[...],
                                      preferred_element_type=jnp.float32)


@jax.jit
def _moe_forward(x, wr_t, gu_all, dn_all):
    B, S, H = x.shape
    T = B * S
    E1, _, twoI = gu_all.shape
    E = E1 - 1
    I = twoI // 2
    top_k = 2

    x2d = x.reshape(T, H)
    n_half = 2
    T_half = T // n_half
    tm = min(512, T_half)
    n_t = T_half // tm

    out, scores = pl.pallas_call(
        functools.partial(_fused_moe_kernel, top_k=top_k, n_exp=E,
                          inter=I, tm=tm),
        out_shape=(
            jax.ShapeDtypeStruct((T, H), jnp.float32),
            jax.ShapeDtypeStruct((T, E), jnp.float32),
        ),
        grid=(n_half, E1, n_t),
        in_specs=[
            pl.BlockSpec((tm, H),
                         lambda th, e, ti: (th * n_t + jnp.where(e == 0, ti, 0),
                                            0)),
            pl.BlockSpec((H, E), lambda th, e, ti: (0, 0)),
            pl.BlockSpec((None, H, twoI),
                         lambda th, e, ti: ((e + E) % E1, 0, 0)),
            pl.BlockSpec((None, I, H),
                         lambda th, e, ti: ((e + E) % E1, 0, 0)),
        ],
        out_specs=(
            pl.BlockSpec((T_half, H), lambda th, e, ti: (th, 0)),
            pl.BlockSpec((T_half, E), lambda th, e, ti: (th, 0)),
        ),
        scratch_shapes=[pltpu.VMEM((T_half, E), jnp.float32),
                        pltpu.VMEM((T_half, H), jnp.bfloat16)],
        compiler_params=pltpu.CompilerParams(
            dimension_semantics=("arbitrary", "arbitrary", "arbitrary"),
            vmem_limit_bytes=64 << 20),
    )(x2d, wr_t, gu_all, dn_all)

    return out, scores.T


def kernel(x, wr_t, gu_all, dn_all):
    return _moe_forward(x, wr_t, gu_all, dn_all)


# speedup vs baseline: 1.0064x; 1.0064x over previous
import functools

import jax
import jax.numpy as jnp
from jax.experimental import pallas as pl
from jax.experimental.pallas import tpu as pltpu


def _fused_moe_kernel(x_ref, wr_ref, gu_ref, dn_ref,
                      out_ref, scores_ref, sc_scr, xb_scr,
                      *, top_k, n_exp, inter, tm):
    e = pl.program_id(1)
    ti = pl.program_id(2)
    rows = pl.ds(ti * tm, tm)
    x = x_ref[...]

    @pl.when(e == 0)
    def _router_and_shared():
        logits = jnp.dot(x, wr_ref[...], preferred_element_type=jnp.float32)
        col = jax.lax.broadcasted_iota(jnp.int32, logits.shape, 1)
        masked = logits
        keep = jnp.zeros(logits.shape, dtype=jnp.bool_)
        for _ in range(top_k):
            m = jnp.max(masked, axis=-1, keepdims=True)
            is_max = masked == m
            first_idx = jnp.min(jnp.where(is_max, col, n_exp),
                                axis=-1, keepdims=True)
            sel = col == first_idx
            keep = jnp.logical_or(keep, sel)
            masked = jnp.where(sel, -jnp.inf, masked)
        scores = jnp.where(keep, jax.nn.sigmoid(logits), 0.0)
        sc_scr[rows, :] = scores
        scores_ref[rows, :] = scores
        xb = x.astype(jnp.bfloat16)
        xb_scr[rows, :] = xb

        gu = jnp.dot(xb, gu_ref[...], preferred_element_type=jnp.float32)
        g = gu[:, :inter]
        u = gu[:, inter:]
        h = g * jax.nn.sigmoid(g) * u
        out_ref[rows, :] = jnp.dot(h, dn_ref[...],
                                   preferred_element_type=jnp.float32)

    @pl.when(e > 0)
    def _routed():
        sc = sc_scr[rows, :]
        col = jax.lax.broadcasted_iota(jnp.int32, sc.shape, 1)
        s = jnp.sum(jnp.where(col == e - 1, sc, 0.0), axis=1, keepdims=True)
        gu = jnp.dot(xb_scr[rows, :], gu_ref[...],
                     preferred_element_type=jnp.float32) * s
        g = gu[:, :inter]
        u = gu[:, inter:]
        h = g * jax.nn.sigmoid(g) * u
        out_ref[rows, :] += jnp.dot(h, dn_ref[...],
                                    preferred_element_type=jnp.float32)


@jax.jit
def _moe_forward(x, wr_t, gu_all, dn_all):
    B, S, H = x.shape
    T = B * S
    E1, _, twoI = gu_all.shape
    E = E1 - 1
    I = twoI // 2
    top_k = 2

    x2d = x.reshape(T, H)
    n_half = 2
    T_half = T // n_half
    tm = min(512, T_half)
    n_t = T_half // tm

    out, scores = pl.pallas_call(
        functools.partial(_fused_moe_kernel, top_k=top_k, n_exp=E,
                          inter=I, tm=tm),
        out_shape=(
            jax.ShapeDtypeStruct((T, H), jnp.float32),
            jax.ShapeDtypeStruct((T, E), jnp.float32),
        ),
        grid=(n_half, E1, n_t),
        in_specs=[
            pl.BlockSpec((tm, H),
                         lambda th, e, ti: (th * n_t + jnp.where(e == 0, ti, 0),
                                            0)),
            pl.BlockSpec((H, E), lambda th, e, ti: (0, 0)),
            pl.BlockSpec((None, H, twoI),
                         lambda th, e, ti: ((e + E) % E1, 0, 0)),
            pl.BlockSpec((None, I, H),
                         lambda th, e, ti: ((e + E) % E1, 0, 0)),
        ],
        out_specs=(
            pl.BlockSpec((T_half, H), lambda th, e, ti: (th, 0)),
            pl.BlockSpec((T_half, E), lambda th, e, ti: (th, 0)),
        ),
        scratch_shapes=[pltpu.VMEM((T_half, E), jnp.float32),
                        pltpu.VMEM((T_half, H), jnp.bfloat16)],
        compiler_params=pltpu.CompilerParams(
            dimension_semantics=("arbitrary", "arbitrary", "arbitrary"),
            vmem_limit_bytes=64 << 20),
    )(x2d, wr_t, gu_all, dn_all)

    return out, scores.T


def kernel(x, wr_t, gu_all, dn_all):
    return _moe_forward(x, wr_t, gu_all, dn_all)
